# Initial kernel scaffold; baseline (speedup 1.0000x reference)
import jax
import jax.numpy as jnp
from jax.experimental import pallas as pl


def kernel(x, edge_index, batch, W1, a_src1, a_dst1, b1, W2, a_src2, a_dst2, b2, fc_w, fc_b):
    raise NotImplementedError("write your pallas kernel here")


# speedup vs baseline: 1.3365x; 1.3365x over previous
import functools

import jax
import jax.numpy as jnp
from jax import lax
from jax.experimental import pallas as pl
from jax.experimental.pallas import tpu as pltpu
from jax.experimental.pallas import tpu_sc as plsc

N = 10000
NPAD = 10240
E = 160000
H = 256
HH = 128
G = 64
NT = 32
EPT = 5120
EPAD = NT * EPT
RB = 256
NRB = NPAD // RB


def _emit_heads(h, a8_ref, hh_ref, as_ref, ad_ref):
    hh_ref[...] = jnp.stack([h[:, :HH], h[:, HH:]], axis=0)
    aa = jnp.dot(h, a8_ref[...], preferred_element_type=jnp.float32,
                 precision=lax.Precision.HIGHEST)
    as_ref[...] = jnp.broadcast_to(aa[:, 0:1], (RB, HH))
    ad_ref[...] = jnp.broadcast_to(aa[:, 1:2], (RB, HH))


def _mm1_body(x_ref, w_ref, a8_ref, hh_ref, as_ref, ad_ref):
    h = jnp.dot(x_ref[...], w_ref[...], preferred_element_type=jnp.float32,
        precision=lax.Precision.HIGHEST)
    _emit_heads(h, a8_ref, hh_ref, as_ref, ad_ref)


def _mm1(x_pad, w, a8):
    return pl.pallas_call(
        _mm1_body,
        grid=(NRB,),
        in_specs=[
            pl.BlockSpec((RB, H), lambda b: (b, 0)),
            pl.BlockSpec((H, H), lambda b: (0, 0)),
            pl.BlockSpec((H, 16), lambda b: (0, 0)),
        ],
        out_specs=[
            pl.BlockSpec((2, RB, HH), lambda b: (0, b, 0)),
            pl.BlockSpec((RB, HH), lambda b: (b, 0)),
            pl.BlockSpec((RB, HH), lambda b: (b, 0)),
        ],
        out_shape=[
            jax.ShapeDtypeStruct((2, NPAD, HH), jnp.float32),
            jax.ShapeDtypeStruct((NPAD, HH), jnp.float32),
            jax.ShapeDtypeStruct((NPAD, HH), jnp.float32),
        ],
    )(x_pad, w, a8)


def _combine(agg_ref, dp_ref, b_ref):
    agg = agg_ref[...]
    aggsum = agg[0] + agg[1]
    z = jnp.concatenate([aggsum[0], aggsum[1]], axis=1)
    d = dp_ref[...]
    dinv = 1.0 / (d[0, :, 0] + d[1, :, 0] + 1e-16)
    return jnp.maximum(z * dinv[:, None] + b_ref[0:1, :], 0.0)


def _mm2_body(agg_ref, dp_ref, b_ref, w_ref, a8_ref, hh_ref, as_ref, ad_ref):
    zz = _combine(agg_ref, dp_ref, b_ref)
    h = jnp.dot(zz, w_ref[...], preferred_element_type=jnp.float32,
        precision=lax.Precision.HIGHEST)
    _emit_heads(h, a8_ref, hh_ref, as_ref, ad_ref)


def _mm2(agg, dparts, b1p, w, a8):
    return pl.pallas_call(
        _mm2_body,
        grid=(NRB,),
        in_specs=[
            pl.BlockSpec((2, 2, RB, HH), lambda b: (0, 0, b, 0)),
            pl.BlockSpec((2, RB, HH), lambda b: (0, b, 0)),
            pl.BlockSpec((8, H), lambda b: (0, 0)),
            pl.BlockSpec((H, H), lambda b: (0, 0)),
            pl.BlockSpec((H, 16), lambda b: (0, 0)),
        ],
        out_specs=[
            pl.BlockSpec((2, RB, HH), lambda b: (0, b, 0)),
            pl.BlockSpec((RB, HH), lambda b: (b, 0)),
            pl.BlockSpec((RB, HH), lambda b: (b, 0)),
        ],
        out_shape=[
            jax.ShapeDtypeStruct((2, NPAD, HH), jnp.float32),
            jax.ShapeDtypeStruct((NPAD, HH), jnp.float32),
            jax.ShapeDtypeStruct((NPAD, HH), jnp.float32),
        ],
    )(agg, dparts, b1p, w, a8)


def _pool_body(agg_ref, dp_ref, b_ref, batch_ref, fcw_ref, fcb_ref,
               out_ref, gmax_ref, gsum_ref, cnt_ref):
    b = pl.program_id(0)

    @pl.when(b == 0)
    def _init():
        gmax_ref[...] = jnp.full((G, H), -3.4e38, jnp.float32)
        gsum_ref[...] = jnp.zeros((G, H), jnp.float32)
        cnt_ref[...] = jnp.zeros((8, G), jnp.float32)

    h = _combine(agg_ref, dp_ref, b_ref)
    bvec = batch_ref[0, 0, :]
    iota_g = lax.broadcasted_iota(jnp.int32, (RB, G), 1)
    oh = (bvec[:, None] == iota_g).astype(jnp.float32)
    gsum_ref[...] += lax.dot_general(
        oh, h, (((0,), (0,)), ((), ())), preferred_element_type=jnp.float32,
        precision=lax.Precision.HIGHEST)
    cnt_ref[0:1, :] += jnp.sum(oh, axis=0)[None, :]

    g_lo = bvec[0]
    g_hi = jnp.minimum(bvec[RB - 1], G - 1)

    def body(g, carry):
        m = jnp.where(bvec[:, None] == g, h, -3.4e38)
        mg = jnp.max(m, axis=0)
        gmax_ref[pl.ds(g, 1), :] = jnp.maximum(gmax_ref[pl.ds(g, 1), :],
                                               mg[None, :])
        return carry

    lax.fori_loop(g_lo, g_hi + 1, body, 0)

    @pl.when(b == pl.num_programs(0) - 1)
    def _fin():
        cnt = cnt_ref[0:1, :][0]
        gmax = jnp.where(cnt[:, None] > 0, gmax_ref[...], 0.0)
        gmean = gsum_ref[...] / jnp.maximum(cnt[:, None], 1.0)
        pooled = jnp.concatenate([gmax, gmean], axis=1)
        res = jnp.sum(pooled * fcw_ref[0:1, :], axis=1) + fcb_ref[0, 0]
        out_ref[...] = jnp.broadcast_to(res[:, None], (G, 128))


def _pool(agg, dparts, b2p, batch3d, fcwp, fcbp):
    return pl.pallas_call(
        _pool_body,
        grid=(NRB,),
        in_specs=[
            pl.BlockSpec((2, 2, RB, HH), lambda b: (0, 0, b, 0)),
            pl.BlockSpec((2, RB, HH), lambda b: (0, b, 0)),
            pl.BlockSpec((8, H), lambda b: (0, 0)),
            pl.BlockSpec((1, 1, RB), lambda b: (b, 0, 0)),
            pl.BlockSpec((8, 2 * H), lambda b: (0, 0)),
            pl.BlockSpec((8, 128), lambda b: (0, 0)),
        ],
        out_specs=pl.BlockSpec((G, 128), lambda b: (0, 0)),
        out_shape=jax.ShapeDtypeStruct((G, 128), jnp.float32),
        scratch_shapes=[
            pltpu.VMEM((G, H), jnp.float32),
            pltpu.VMEM((G, H), jnp.float32),
            pltpu.VMEM((8, G), jnp.float32),
        ],
    )(agg, dparts, b2p, batch3d, fcwp, fcbp)


@functools.cache
def _sc_mesh():
    return plsc.VectorSubcoreMesh(core_axis_name="c", subcore_axis_name="s",
                                  num_cores=2, num_subcores=16)


_EREAL = E // NT
_NSL = NPAD // 16
_CH = 64
_NCH = EPT // _CH
_NH = 2560
_SHROWS = 2688


def _edge_body(srct, dst2, dstf, as128, ad128, hh, z2, dp_out, agg_out,
               exx_out, src_v, dst2_v, dst_v, ar_v, br_v, exd_v, exxc_v,
               idxb_v, shd):
    c = lax.axis_index("c")
    s = lax.axis_index("s")
    wid = s * 2 + c
    NH = _NH
    ZR = _SHROWS // 16
    pltpu.sync_copy(srct.at[wid], src_v)
    pltpu.sync_copy(dst2.at[wid], dst2_v)
    pltpu.sync_copy(dstf.at[wid], dst_v)

    def chunk0(j, carry):
        pltpu.sync_copy(as128.at[src_v.at[pl.ds(j * _CH, _CH)]], ar_v)
        pltpu.sync_copy(ad128.at[dst_v.at[pl.ds(j * _CH, _CH)]], br_v)

        def row(r, carry2):
            e = ar_v[r, pl.ds(0, 16)] + br_v[r, pl.ds(0, 16)]
            e = jnp.where(e >= 0, e, 0.2 * e)
            exr = jnp.exp(e)
            exr = jnp.where(j * _CH + r < _EREAL, exr, 0.0)
            exxc_v[r, :] = exr
            return carry2

        lax.fori_loop(0, _CH, row, 0)
        pltpu.sync_copy(exxc_v, exx_out.at[wid, j])
        return carry

    lax.fori_loop(0, _NCH, chunk0, 0)

    def mkidx(j, base):
        def g16(g, carry):
            d16 = dst_v[pl.ds(j * _CH + g * 16, 16)]
            rel = d16 - base
            ok = (rel >= 0) & (rel < NH)
            idxb_v[pl.ds(g * 16, 16)] = jnp.where(ok, rel, NH)
            return carry

        lax.fori_loop(0, _CH // 16, g16, 0)

    def zero_shd():
        pltpu.sync_copy(z2.at[pl.ds(s * ZR, ZR)], shd.at[pl.ds(s * ZR, ZR)])
        plsc.subcore_barrier()

    for p in range(4):
        FR = min(NH, NPAD - p * NH) // 16
        zero_shd()

        def chunk1(j, carry):
            pltpu.sync_copy(exx_out.at[wid, j], exxc_v)

            def row(r, carry2):
                exr = exxc_v[r, :]
                for k in range(HH // 16):
                    exd_v[r, pl.ds(k * 16, 16)] = exr
                return carry2

            lax.fori_loop(0, _CH, row, 0)
            mkidx(j, p * NH)
            pltpu.sync_copy(exd_v, shd.at[idxb_v], add=True)
            return carry

        lax.fori_loop(0, _NCH, chunk1, 0)
        plsc.subcore_barrier()
        pltpu.sync_copy(shd.at[pl.ds(s * FR, FR)],
                        dp_out.at[c, pl.ds(p * NH + s * FR, FR)])
        plsc.subcore_barrier()

    for half in range(2):
        for p in range(4):
            FR = min(NH, NPAD - p * NH) // 16
            zero_shd()

            def chunk2(j, carry):
                pltpu.sync_copy(hh.at[half].at[src_v.at[pl.ds(j * _CH, _CH)]],
                                ar_v)
                pltpu.sync_copy(exx_out.at[wid, j], exxc_v)

                def row2(r, carry2):
                    wb = exxc_v[r, :]
                    for k in range(HH // 16):
                        ksl = pl.ds(k * 16, 16)
                        ar_v[r, ksl] = ar_v[r, ksl] * wb
                    return carry2

                lax.fori_loop(0, _CH, row2, 0)
                mkidx(j, p * NH)
                pltpu.sync_copy(ar_v, shd.at[idxb_v], add=True)
                return carry

            lax.fori_loop(0, _NCH, chunk2, 0)
            plsc.subcore_barrier()
            pltpu.sync_copy(shd.at[pl.ds(s * FR, FR)],
                            agg_out.at[c, half, pl.ds(p * NH + s * FR, FR)])
            plsc.subcore_barrier()


def _edge_layer(as128, ad128, hh, srct, dst2, dstf, z2):
    f = pl.kernel(
        _edge_body,
        out_type=[jax.ShapeDtypeStruct((2, NPAD, HH), jnp.float32),
                  jax.ShapeDtypeStruct((2, 2, NPAD, HH), jnp.float32),
                  jax.ShapeDtypeStruct((NT, _NCH, _CH, 16), jnp.float32)],
        mesh=_sc_mesh(),
        scratch_types=[
            pltpu.VMEM((EPT,), jnp.int32),
            pltpu.VMEM((_NCH, _CH), jnp.int32),
            pltpu.VMEM((EPT,), jnp.int32),
            pltpu.VMEM((_CH, HH), jnp.float32),
            pltpu.VMEM((_CH, HH), jnp.float32),
            pltpu.VMEM((_CH, HH), jnp.float32),
            pltpu.VMEM((_CH, 16), jnp.float32),
            pltpu.VMEM((_CH,), jnp.int32),
            pltpu.VMEM_SHARED((_SHROWS, HH), jnp.float32),
        ],
    )
    dp, agg, _ = f(srct, dst2, dstf, as128, ad128, hh, z2)
    return dp, agg


def kernel(x, edge_index, batch, W1, a_src1, a_dst1, b1,
           W2, a_src2, a_dst2, b2, fc_w, fc_b):
    f32 = jnp.float32
    x_pad = jnp.zeros((NPAD, H), f32).at[:N].set(x)

    def pack_a8(a_src, a_dst):
        a8 = jnp.zeros((H, 16), f32)
        return a8.at[:, 0].set(a_src).at[:, 1].set(a_dst)

    def pack_row8(v):
        return jnp.zeros((8, H), f32).at[0].set(v)

    src = edge_index[0].reshape(NT, E // NT)
    dst = edge_index[1].reshape(NT, E // NT)
    pad = EPT - E // NT
    srct = jnp.pad(src, ((0, 0), (0, pad)), constant_values=NPAD - 1)
    dstf = jnp.pad(dst, ((0, 0), (0, pad)), constant_values=NPAD - 1)
    dst2 = dstf.reshape(NT, _NCH, _CH)

    z2 = jnp.zeros((NPAD, HH), f32)

    batch_pad = jnp.full((NPAD,), G, jnp.int32).at[:N].set(batch)
    batch3d = batch_pad.reshape(NRB, 1, RB)
    fcwp = jnp.zeros((8, 2 * H), f32).at[0].set(fc_w[:, 0])
    fcbp = jnp.zeros((8, 128), f32).at[0, 0].set(fc_b[0])

    hh1, as1, ad1 = _mm1(x_pad, W1, pack_a8(a_src1, a_dst1))
    dp1, agg1 = _edge_layer(as1, ad1, hh1, srct, dst2, dstf, z2)
    hh2, as2, ad2 = _mm2(agg1, dp1, pack_row8(b1), W2, pack_a8(a_src2, a_dst2))
    dp2, agg2 = _edge_layer(as2, ad2, hh2, srct, dst2, dstf, z2)
    out = _pool(agg2, dp2, pack_row8(b2), batch3d, fcwp, fcbp)
    return out[:, :1]
